# Initial kernel scaffold; baseline (speedup 1.0000x reference)
import jax
import jax.numpy as jnp
from jax.experimental import pallas as pl


def kernel(x, edge_index, edge_attr, node_ids, W1, b1, W2, b2):
    raise NotImplementedError("write your pallas kernel here")


# speedup vs baseline: 3.4485x; 3.4485x over previous
import functools

import jax
import jax.numpy as jnp
from jax import lax
from jax.experimental import pallas as pl
from jax.experimental.pallas import tpu as pltpu
from jax.experimental.pallas import tpu_sc as plsc

N = 10000
E = 160000
D = 128
K = 4
TK = 2 * K
N_PAD = 10240
CH = 128
EP = 327680
ROWS = EP // CH
TILES = 32
RPT = ROWS // TILES
RPS = N_PAD // 16
BN = 400
GRID = N // BN


def _tab_x_body(x_ref, w_ref, b_ref, o_ref):
    xb = x_ref[...]
    for j in range(TK):
        o_ref[j] = jnp.dot(xb, w_ref[j], preferred_element_type=jnp.float32) + b_ref[j]


def _tab_p_body(p_ref, w_ref, b_ref, o_ref):
    h = jnp.maximum(p_ref[0] + p_ref[1], 0.0)
    for j in range(TK):
        o_ref[j] = jnp.dot(h, w_ref[j], preferred_element_type=jnp.float32) + b_ref[j]


def _sum_body(p_ref, o_ref):
    o_ref[...] = p_ref[0] + p_ref[1]


def _tables_from_x(x, w, b):
    return pl.pallas_call(
        _tab_x_body,
        grid=(GRID,),
        in_specs=[
            pl.BlockSpec((BN, D), lambda i: (i, 0)),
            pl.BlockSpec((TK, D, D), lambda i: (0, 0, 0)),
            pl.BlockSpec((TK, D), lambda i: (0, 0)),
        ],
        out_specs=pl.BlockSpec((TK, BN, D), lambda i: (0, i, 0)),
        out_shape=jax.ShapeDtypeStruct((TK, N, D), jnp.float32),
    )(x, w, b)


def _tables_from_partials(p, w, b):
    return pl.pallas_call(
        _tab_p_body,
        grid=(GRID,),
        in_specs=[
            pl.BlockSpec((2, BN, D), lambda i: (0, i, 0)),
            pl.BlockSpec((TK, D, D), lambda i: (0, 0, 0)),
            pl.BlockSpec((TK, D), lambda i: (0, 0)),
        ],
        out_specs=pl.BlockSpec((TK, BN, D), lambda i: (0, i, 0)),
        out_shape=jax.ShapeDtypeStruct((TK, N, D), jnp.float32),
    )(p, w, b)


def _sum_partials(p):
    return pl.pallas_call(
        _sum_body,
        grid=(GRID,),
        in_specs=[pl.BlockSpec((2, BN, D), lambda i: (0, i, 0))],
        out_specs=pl.BlockSpec((BN, D), lambda i: (i, 0)),
        out_shape=jax.ShapeDtypeStruct((N, D), jnp.float32),
    )(p)


def _sc_body(t_hbm, gidx_hbm, sidx_hbm, z_hbm, out_hbm,
             gidx_v, sidx_v, rows_v, acc, sem):
    cid = lax.axis_index("c")
    sid = lax.axis_index("s")
    tid = cid * 16 + sid
    pltpu.sync_copy(gidx_hbm.at[pl.ds(tid * RPT, RPT)], gidx_v)
    pltpu.sync_copy(sidx_hbm.at[pl.ds(tid * RPT, RPT)], sidx_v)
    pltpu.sync_copy(z_hbm, acc.at[pl.ds(sid * RPS, RPS)])
    plsc.subcore_barrier()

    def step(t, carry):
        pltpu.async_copy(t_hbm.at[gidx_v.at[t]], rows_v, sem).wait()
        pltpu.sync_copy(rows_v, acc.at[sidx_v.at[t]], add=True)
        return carry

    lax.fori_loop(0, RPT, step, 0)
    plsc.subcore_barrier()

    def wstep(kk, carry):
        r = sid * RPS + kk * CH
        pltpu.sync_copy(acc.at[pl.ds(r, CH)], rows_v)
        pltpu.sync_copy(rows_v, out_hbm.at[pl.ds(cid * N_PAD + r, CH)])
        return carry

    lax.fori_loop(0, RPS // CH, wstep, 0)


@functools.cache
def _sc_gather_scatter():
    return pl.kernel(
        _sc_body,
        mesh=plsc.VectorSubcoreMesh(core_axis_name="c", subcore_axis_name="s"),
        out_type=jax.ShapeDtypeStruct((2 * N_PAD, D), jnp.float32),
        scratch_types=[
            pltpu.VMEM((RPT, CH), jnp.int32),
            pltpu.VMEM((RPT, CH), jnp.int32),
            pltpu.VMEM((CH, D), jnp.float32),
            pltpu.VMEM_SHARED((N_PAD, D), jnp.float32),
            pltpu.SemaphoreType.DMA,
        ],
    )


def kernel(x, edge_index, edge_attr, node_ids, W1, b1, W2, b2):
    src = edge_index[0]
    dst = edge_index[1]
    et = edge_attr

    gidx = jnp.concatenate([et * N + src, (K + et) * N + dst])
    sidx = jnp.concatenate([dst, dst])
    pad = EP - 2 * E
    gidx = jnp.concatenate([gidx, jnp.zeros((pad,), jnp.int32)]).reshape(ROWS, CH)
    sidx = jnp.concatenate([sidx, jnp.full((pad,), N_PAD - 1, jnp.int32)]).reshape(ROWS, CH)
    zrows = jnp.zeros((RPS, D), jnp.float32)

    Wc1 = jnp.concatenate([W1[:, :D, :], W1[:, D:, :]], axis=0)
    bc1 = jnp.concatenate([jnp.zeros_like(b1), b1], axis=0)
    Wc2 = jnp.concatenate([W2[:, :D, :], W2[:, D:, :]], axis=0)
    bc2 = jnp.concatenate([jnp.zeros_like(b2), b2], axis=0)

    t1 = _tables_from_x(x, Wc1, bc1)
    p1 = _sc_gather_scatter()(t1.reshape(TK * N, D), gidx, sidx, zrows)
    t2 = _tables_from_partials(p1.reshape(2, N_PAD, D), Wc2, bc2)
    p2 = _sc_gather_scatter()(t2.reshape(TK * N, D), gidx, sidx, zrows)
    return _sum_partials(p2.reshape(2, N_PAD, D))


# speedup vs baseline: 8.1434x; 2.3614x over previous
import functools

import jax
import jax.numpy as jnp
from jax import lax
from jax.experimental import pallas as pl
from jax.experimental.pallas import tpu as pltpu
from jax.experimental.pallas import tpu_sc as plsc

N = 10000
E = 160000
D = 128
K = 4
TK = 2 * K
N_PAD = 10240
CH = 128
EP = 327680
ROWS = EP // CH
TILES = 32
RPT = ROWS // TILES
RPS = N_PAD // 16
BN = 400
GRID = N // BN


def _tab_x_body(x_ref, w_ref, b_ref, o_ref):
    xb = x_ref[...]
    for j in range(TK):
        o_ref[j] = jnp.dot(xb, w_ref[j], preferred_element_type=jnp.float32) + b_ref[j]


def _tab_p_body(p_ref, w_ref, b_ref, o_ref):
    h = jnp.maximum(p_ref[0] + p_ref[1], 0.0)
    for j in range(TK):
        o_ref[j] = jnp.dot(h, w_ref[j], preferred_element_type=jnp.float32) + b_ref[j]


def _sum_body(p_ref, o_ref):
    o_ref[...] = p_ref[0] + p_ref[1]


def _tables_from_x(x, w, b):
    return pl.pallas_call(
        _tab_x_body,
        grid=(GRID,),
        in_specs=[
            pl.BlockSpec((BN, D), lambda i: (i, 0)),
            pl.BlockSpec((TK, D, D), lambda i: (0, 0, 0)),
            pl.BlockSpec((TK, D), lambda i: (0, 0)),
        ],
        out_specs=pl.BlockSpec((TK, BN, D), lambda i: (0, i, 0)),
        out_shape=jax.ShapeDtypeStruct((TK, N, D), jnp.float32),
    )(x, w, b)


def _tables_from_partials(p, w, b):
    return pl.pallas_call(
        _tab_p_body,
        grid=(GRID,),
        in_specs=[
            pl.BlockSpec((2, BN, D), lambda i: (0, i, 0)),
            pl.BlockSpec((TK, D, D), lambda i: (0, 0, 0)),
            pl.BlockSpec((TK, D), lambda i: (0, 0)),
        ],
        out_specs=pl.BlockSpec((TK, BN, D), lambda i: (0, i, 0)),
        out_shape=jax.ShapeDtypeStruct((TK, N, D), jnp.float32),
    )(p, w, b)


def _sum_partials(p):
    return pl.pallas_call(
        _sum_body,
        grid=(GRID,),
        in_specs=[pl.BlockSpec((2, BN, D), lambda i: (0, i, 0))],
        out_specs=pl.BlockSpec((BN, D), lambda i: (i, 0)),
        out_shape=jax.ShapeDtypeStruct((N, D), jnp.float32),
    )(p)


def _sc_body(t_hbm, gidx_hbm, sidx_hbm, z_hbm, out_hbm,
             gidx_v, sidx_v, rows_v, acc, sem):
    cid = lax.axis_index("c")
    sid = lax.axis_index("s")
    tid = cid * 16 + sid
    pltpu.sync_copy(gidx_hbm.at[pl.ds(tid * RPT, RPT)], gidx_v)
    pltpu.sync_copy(sidx_hbm.at[pl.ds(tid * RPT, RPT)], sidx_v)
    pltpu.sync_copy(z_hbm, acc.at[pl.ds(sid * RPS, RPS)])
    plsc.subcore_barrier()

    def step(t, carry):
        pltpu.async_copy(t_hbm.at[gidx_v.at[t]], rows_v, sem).wait()
        pltpu.sync_copy(rows_v, acc.at[sidx_v.at[t]], add=True)
        return carry

    lax.fori_loop(0, RPT, step, 0)
    plsc.subcore_barrier()

    def wstep(kk, carry):
        r = sid * RPS + kk * CH
        pltpu.sync_copy(acc.at[pl.ds(r, CH)], rows_v)
        pltpu.sync_copy(rows_v, out_hbm.at[pl.ds(cid * N_PAD + r, CH)])
        return carry

    lax.fori_loop(0, RPS // CH, wstep, 0)


@functools.cache
def _sc_gather_scatter():
    return pl.kernel(
        _sc_body,
        mesh=plsc.VectorSubcoreMesh(core_axis_name="c", subcore_axis_name="s"),
        out_type=jax.ShapeDtypeStruct((2 * N_PAD, D), jnp.float32),
        scratch_types=[
            pltpu.VMEM((RPT, CH), jnp.int32),
            pltpu.VMEM((RPT, CH), jnp.int32),
            pltpu.VMEM((CH, D), jnp.float32),
            pltpu.VMEM_SHARED((N_PAD, D), jnp.float32),
            pltpu.SemaphoreType.DMA,
        ],
    )


def kernel(x, edge_index, edge_attr, node_ids, W1, b1, W2, b2):
    src = edge_index[0]
    dst = edge_index[1]
    et = edge_attr

    gidx = jnp.concatenate([et * N + src, (K + et) * N + dst])
    sidx = jnp.concatenate([dst, dst])
    pad = EP - 2 * E
    pad_g = jnp.arange(pad, dtype=jnp.int32) % 128
    pad_s = N + jnp.arange(pad, dtype=jnp.int32) % (N_PAD - N)
    gidx = jnp.concatenate([gidx, pad_g]).reshape(ROWS, CH)
    sidx = jnp.concatenate([sidx, pad_s]).reshape(ROWS, CH)
    zrows = jnp.zeros((RPS, D), jnp.float32)

    Wc1 = jnp.concatenate([W1[:, :D, :], W1[:, D:, :]], axis=0)
    bc1 = jnp.concatenate([jnp.zeros_like(b1), b1], axis=0)
    Wc2 = jnp.concatenate([W2[:, :D, :], W2[:, D:, :]], axis=0)
    bc2 = jnp.concatenate([jnp.zeros_like(b2), b2], axis=0)

    t1 = _tables_from_x(x, Wc1, bc1)
    p1 = _sc_gather_scatter()(t1.reshape(TK * N, D), gidx, sidx, zrows)
    t2 = _tables_from_partials(p1.reshape(2, N_PAD, D), Wc2, bc2)
    p2 = _sc_gather_scatter()(t2.reshape(TK * N, D), gidx, sidx, zrows)
    return _sum_partials(p2.reshape(2, N_PAD, D))


# speedup vs baseline: 11.6970x; 1.4364x over previous
import functools

import jax
import jax.numpy as jnp
from jax import lax
from jax.experimental import pallas as pl
from jax.experimental.pallas import tpu as pltpu
from jax.experimental.pallas import tpu_sc as plsc

N = 10000
E = 160000
D = 128
K = 4
TK = 2 * K
N_PAD = 10240
CH = 128
EP = 327680
ROWS = EP // CH
TILES = 32
RPT = ROWS // TILES
RPS = N_PAD // 16
BN = 400
GRID = N // BN


def _tab_x_body(x_ref, w_ref, b_ref, o_ref):
    xb = x_ref[...]
    for j in range(TK):
        o_ref[j] = jnp.dot(xb, w_ref[j], preferred_element_type=jnp.float32) + b_ref[j]


def _tab_p_body(p_ref, w_ref, b_ref, o_ref):
    h = jnp.maximum(p_ref[0] + p_ref[1], 0.0)
    for j in range(TK):
        o_ref[j] = jnp.dot(h, w_ref[j], preferred_element_type=jnp.float32) + b_ref[j]


def _sum_body(p_ref, o_ref):
    o_ref[...] = p_ref[0] + p_ref[1]


def _tables_from_x(x, w, b):
    return pl.pallas_call(
        _tab_x_body,
        grid=(GRID,),
        in_specs=[
            pl.BlockSpec((BN, D), lambda i: (i, 0)),
            pl.BlockSpec((TK, D, D), lambda i: (0, 0, 0)),
            pl.BlockSpec((TK, D), lambda i: (0, 0)),
        ],
        out_specs=pl.BlockSpec((TK, BN, D), lambda i: (0, i, 0)),
        out_shape=jax.ShapeDtypeStruct((TK, N, D), jnp.float32),
    )(x, w, b)


def _tables_from_partials(p, w, b):
    return pl.pallas_call(
        _tab_p_body,
        grid=(GRID,),
        in_specs=[
            pl.BlockSpec((2, BN, D), lambda i: (0, i, 0)),
            pl.BlockSpec((TK, D, D), lambda i: (0, 0, 0)),
            pl.BlockSpec((TK, D), lambda i: (0, 0)),
        ],
        out_specs=pl.BlockSpec((TK, BN, D), lambda i: (0, i, 0)),
        out_shape=jax.ShapeDtypeStruct((TK, N, D), jnp.float32),
    )(p, w, b)


def _sum_partials(p):
    return pl.pallas_call(
        _sum_body,
        grid=(GRID,),
        in_specs=[pl.BlockSpec((2, BN, D), lambda i: (0, i, 0))],
        out_specs=pl.BlockSpec((BN, D), lambda i: (i, 0)),
        out_shape=jax.ShapeDtypeStruct((N, D), jnp.float32),
    )(p)


def _sc_body(t_hbm, gidx_hbm, sidx_hbm, z_hbm, out_hbm,
             gidx_v, s0, s1, r0, r1, acc, g0, g1, e0, e1):
    cid = lax.axis_index("c")
    sid = lax.axis_index("s")
    tid = cid * 16 + sid
    base = tid * RPT
    pltpu.sync_copy(gidx_hbm.at[pl.ds(base, RPT)], gidx_v)
    pltpu.sync_copy(z_hbm, acc.at[pl.ds(sid * RPS, RPS)])
    plsc.subcore_barrier()

    pltpu.async_copy(t_hbm.at[gidx_v.at[0]], r0, g0)
    pltpu.async_copy(sidx_hbm.at[pl.ds(base, 1)], s0, e0)
    pltpu.async_copy(t_hbm.at[gidx_v.at[1]], r1, g1)
    pltpu.async_copy(sidx_hbm.at[pl.ds(base + 1, 1)], s1, e1)

    def step(j, carry):
        t0 = 2 * j
        t1 = t0 + 1
        pltpu.make_async_copy(t_hbm.at[gidx_v.at[t0]], r0, g0).wait()
        pltpu.make_async_copy(sidx_hbm.at[pl.ds(base + t0, 1)], s0, e0).wait()
        pltpu.sync_copy(r0, acc.at[s0.at[0]], add=True)

        @pl.when(j < RPT // 2 - 1)
        def _():
            pltpu.async_copy(t_hbm.at[gidx_v.at[t0 + 2]], r0, g0)
            pltpu.async_copy(sidx_hbm.at[pl.ds(base + t0 + 2, 1)], s0, e0)

        pltpu.make_async_copy(t_hbm.at[gidx_v.at[t1]], r1, g1).wait()
        pltpu.make_async_copy(sidx_hbm.at[pl.ds(base + t1, 1)], s1, e1).wait()
        pltpu.sync_copy(r1, acc.at[s1.at[0]], add=True)

        @pl.when(j < RPT // 2 - 1)
        def _():
            pltpu.async_copy(t_hbm.at[gidx_v.at[t1 + 2]], r1, g1)
            pltpu.async_copy(sidx_hbm.at[pl.ds(base + t1 + 2, 1)], s1, e1)

        return carry

    lax.fori_loop(0, RPT // 2, step, 0)
    plsc.subcore_barrier()

    def wstep(kk, carry):
        r = sid * RPS + kk * CH
        pltpu.sync_copy(acc.at[pl.ds(r, CH)], r0)
        pltpu.sync_copy(r0, out_hbm.at[pl.ds(cid * N_PAD + r, CH)])
        return carry

    lax.fori_loop(0, RPS // CH, wstep, 0)


@functools.cache
def _sc_gather_scatter():
    return pl.kernel(
        _sc_body,
        mesh=plsc.VectorSubcoreMesh(core_axis_name="c", subcore_axis_name="s"),
        out_type=jax.ShapeDtypeStruct((2 * N_PAD, D), jnp.float32),
        scratch_types=[
            pltpu.VMEM((RPT, CH), jnp.int32),
            pltpu.VMEM((1, CH), jnp.int32),
            pltpu.VMEM((1, CH), jnp.int32),
            pltpu.VMEM((CH, D), jnp.float32),
            pltpu.VMEM((CH, D), jnp.float32),
            pltpu.VMEM_SHARED((N_PAD, D), jnp.float32),
            pltpu.SemaphoreType.DMA,
            pltpu.SemaphoreType.DMA,
            pltpu.SemaphoreType.DMA,
            pltpu.SemaphoreType.DMA,
        ],
    )


def kernel(x, edge_index, edge_attr, node_ids, W1, b1, W2, b2):
    src = edge_index[0]
    dst = edge_index[1]
    et = edge_attr

    gidx = jnp.concatenate([et * N + src, (K + et) * N + dst])
    sidx = jnp.concatenate([dst, dst])
    pad = EP - 2 * E
    pad_g = jnp.arange(pad, dtype=jnp.int32) % 128
    pad_s = N + jnp.arange(pad, dtype=jnp.int32) % (N_PAD - N)
    gidx = jnp.concatenate([gidx, pad_g]).reshape(ROWS, CH)
    sidx = jnp.concatenate([sidx, pad_s]).reshape(ROWS, CH)
    zrows = jnp.zeros((RPS, D), jnp.float32)

    Wc1 = jnp.concatenate([W1[:, :D, :], W1[:, D:, :]], axis=0)
    bc1 = jnp.concatenate([jnp.zeros_like(b1), b1], axis=0)
    Wc2 = jnp.concatenate([W2[:, :D, :], W2[:, D:, :]], axis=0)
    bc2 = jnp.concatenate([jnp.zeros_like(b2), b2], axis=0)

    t1 = _tables_from_x(x, Wc1, bc1)
    p1 = _sc_gather_scatter()(t1.reshape(TK * N, D), gidx, sidx, zrows)
    t2 = _tables_from_partials(p1.reshape(2, N_PAD, D), Wc2, bc2)
    p2 = _sc_gather_scatter()(t2.reshape(TK * N, D), gidx, sidx, zrows)
    return _sum_partials(p2.reshape(2, N_PAD, D))
